# baseline (speedup 1.0000x reference)
import functools

import jax
import jax.numpy as jnp
from jax import lax
from jax.experimental import pallas as pl
from jax.experimental.pallas import tpu as pltpu
from jax.experimental.pallas import tpu_sc as plsc

NC = 2
NS = 16
NW = NC * NS
LANES = 16
B = 128


def _fill_vec(ref, val, n):
    v = jnp.full((LANES,), val, dtype=ref.dtype)

    def body(i, c):
        ref[pl.ds(i * LANES, LANES)] = v
        return c

    lax.fori_loop(0, n // LANES, body, 0)


def _deg_kernel(n_pad, cpw, zs, dst_hbm, degp_hbm, idx_v, ones_v, z_v, deg_sh,
                sem):
    c = lax.axis_index("c")
    s = lax.axis_index("s")
    wid = s * NC + c

    _fill_vec(ones_v, 1.0, B)
    _fill_vec(z_v, 0.0, zs)
    pltpu.sync_copy(z_v, deg_sh.at[pl.ds(s * zs, zs)])
    plsc.subcore_barrier()

    gbase = wid * cpw

    def chunk(ci, carry):
        pltpu.sync_copy(dst_hbm.at[gbase + ci, 1], idx_v)
        pltpu.sync_copy(ones_v, deg_sh.at[idx_v], add=True)
        return carry

    lax.fori_loop(0, cpw, chunk, 0)
    plsc.subcore_barrier()
    pltpu.sync_copy(deg_sh.at[pl.ds(s * zs, zs)],
                    degp_hbm.at[pl.ds(c * n_pad + s * zs, zs)])


def _edge_kernel(n_pad, cpw, zs, d, ei_hbm, xs_hbm, outp_hbm,
                 ei_a, ei_b, rows_a, rows_b, z_v, out_sh, ga, gb, sa, sb):
    c = lax.axis_index("c")
    s = lax.axis_index("s")
    wid = s * NC + c

    def zrow(i, carry):
        def zcol(j, cc):
            z_v[i, pl.ds(j * LANES, LANES)] = jnp.zeros((LANES,), jnp.float32)
            return cc

        lax.fori_loop(0, d // LANES, zcol, 0)
        return carry

    lax.fori_loop(0, 64, zrow, 0)

    def zcopy(t, carry):
        pltpu.sync_copy(z_v, out_sh.at[pl.ds(s * zs + t * 64, 64)])
        return carry

    lax.fori_loop(0, zs // 64, zcopy, 0)
    plsc.subcore_barrier()

    gbase = wid * cpw

    def g_start(ei, rows, sem):
        pltpu.async_copy(xs_hbm.at[ei.at[0]], rows, sem)

    def g_wait(ei, rows, sem):
        pltpu.make_async_copy(xs_hbm.at[ei.at[0]], rows, sem).wait()

    def s_start(ei, rows, sem):
        pltpu.async_copy(rows, out_sh.at[ei.at[1]], sem, add=True)

    def s_wait(ei, rows, sem):
        pltpu.make_async_copy(rows, out_sh.at[ei.at[1]], sem).wait()

    pltpu.sync_copy(ei_hbm.at[gbase], ei_a)
    g_start(ei_a, rows_a, ga)
    pltpu.sync_copy(ei_hbm.at[gbase + 1], ei_b)
    g_start(ei_b, rows_b, gb)
    g_wait(ei_a, rows_a, ga)
    s_start(ei_a, rows_a, sa)

    def body(t, carry):
        g0 = gbase + 2 * t
        s_wait(ei_a, rows_a, sa)
        pltpu.sync_copy(ei_hbm.at[g0], ei_a)
        g_start(ei_a, rows_a, ga)
        g_wait(ei_b, rows_b, gb)
        s_start(ei_b, rows_b, sb)
        s_wait(ei_b, rows_b, sb)
        pltpu.sync_copy(ei_hbm.at[g0 + 1], ei_b)
        g_start(ei_b, rows_b, gb)
        g_wait(ei_a, rows_a, ga)
        s_start(ei_a, rows_a, sa)
        return carry

    lax.fori_loop(1, cpw // 2, body, 0)

    g_wait(ei_b, rows_b, gb)
    s_start(ei_b, rows_b, sb)
    s_wait(ei_a, rows_a, sa)
    s_wait(ei_b, rows_b, sb)

    plsc.subcore_barrier()
    pltpu.sync_copy(out_sh.at[pl.ds(s * zs, zs)],
                    outp_hbm.at[pl.ds(c * n_pad + s * zs, zs)])


def _dis_from_parts(dp_ref):
    deg = dp_ref[0, :] + dp_ref[1, :]
    return jnp.where(deg > 0, lax.rsqrt(jnp.maximum(deg, 1.0)), 0.0)


def _scale_kernel(dp_ref, x_ref, xs_ref):
    dis = _dis_from_parts(dp_ref)
    xs_ref[...] = x_ref[...] * dis[:, None]


def _combine_kernel(p_ref, dp_ref, o_ref):
    dis = _dis_from_parts(dp_ref)
    o_ref[...] = (p_ref[0] + p_ref[1]) * dis[:, None]


@jax.jit
def kernel(x, edge_index):
    n, d = x.shape
    e = edge_index.shape[1]

    n_pad = ((n + NS * LANES - 1) // (NS * LANES)) * (NS * LANES)
    zs = n_pad // NS
    cpw = -(-e // (NW * B))
    cpw += cpw % 2
    e_pad = cpw * B * NW
    sac = n_pad - 1

    ei = (
        jnp.full((2, e_pad), sac, jnp.int32)
        .at[:, :e].set(edge_index)
        .reshape(2, NW * cpw, B)
        .transpose(1, 0, 2)
    )
    x_pad = jnp.zeros((n_pad, d), x.dtype).at[:n].set(x)

    mesh = plsc.VectorSubcoreMesh(core_axis_name="c", subcore_axis_name="s",
                                  num_cores=NC, num_subcores=NS)

    deg_parts = pl.kernel(
        functools.partial(_deg_kernel, n_pad, cpw, zs),
        out_type=jax.ShapeDtypeStruct((NC * n_pad,), jnp.float32),
        mesh=mesh,
        scratch_types=[
            pltpu.VMEM((B,), jnp.int32),
            pltpu.VMEM((B,), jnp.float32),
            pltpu.VMEM((zs,), jnp.float32),
            pltpu.VMEM_SHARED((n_pad,), jnp.float32),
            pltpu.SemaphoreType.DMA,
        ],
    )(ei)
    deg_parts = deg_parts.reshape(NC, n_pad)

    rb = 1024
    grid = n_pad // rb
    xs = pl.pallas_call(
        _scale_kernel,
        grid=(grid,),
        in_specs=[
            pl.BlockSpec((NC, rb), lambda i: (0, i)),
            pl.BlockSpec((rb, d), lambda i: (i, 0)),
        ],
        out_specs=pl.BlockSpec((rb, d), lambda i: (i, 0)),
        out_shape=jax.ShapeDtypeStruct((n_pad, d), jnp.float32),
    )(deg_parts, x_pad)

    out_parts = pl.kernel(
        functools.partial(_edge_kernel, n_pad, cpw, zs, d),
        out_type=jax.ShapeDtypeStruct((NC * n_pad, d), jnp.float32),
        mesh=mesh,
        scratch_types=[
            pltpu.VMEM((2, B), jnp.int32),
            pltpu.VMEM((2, B), jnp.int32),
            pltpu.VMEM((B, d), jnp.float32),
            pltpu.VMEM((B, d), jnp.float32),
            pltpu.VMEM((64, d), jnp.float32),
            pltpu.VMEM_SHARED((n_pad, d), jnp.float32),
            pltpu.SemaphoreType.DMA,
            pltpu.SemaphoreType.DMA,
            pltpu.SemaphoreType.DMA,
            pltpu.SemaphoreType.DMA,
        ],
    )(ei, xs)
    out_parts = out_parts.reshape(NC, n_pad, d)

    out_pad = pl.pallas_call(
        _combine_kernel,
        grid=(grid,),
        in_specs=[
            pl.BlockSpec((NC, rb, d), lambda i: (0, i, 0)),
            pl.BlockSpec((NC, rb), lambda i: (0, i)),
        ],
        out_specs=pl.BlockSpec((rb, d), lambda i: (i, 0)),
        out_shape=jax.ShapeDtypeStruct((n_pad, d), jnp.float32),
    )(out_parts, deg_parts)

    return out_pad[:n]


# speedup vs baseline: 1.0773x; 1.0773x over previous
import functools

import jax
import jax.numpy as jnp
from jax import lax
from jax.experimental import pallas as pl
from jax.experimental.pallas import tpu as pltpu
from jax.experimental.pallas import tpu_sc as plsc

NC = 2
NS = 16
NW = NC * NS
LANES = 16
B = 128
BIAS_NUM = 63
BIAS_DEN = 80


def _fill_vec(ref, val, n):
    v = jnp.full((LANES,), val, dtype=ref.dtype)

    def body(i, c):
        ref[pl.ds(i * LANES, LANES)] = v
        return c

    lax.fori_loop(0, n // LANES, body, 0)


def _deg_kernel(n_pad, cpw, zs, dst_hbm, degp_hbm, idx_v, ones_v, z_v, deg_sh,
                sem):
    c = lax.axis_index("c")
    s = lax.axis_index("s")
    wid = s * NC + c

    _fill_vec(ones_v, 1.0, B)
    _fill_vec(z_v, 0.0, zs)
    pltpu.sync_copy(z_v, deg_sh.at[pl.ds(s * zs, zs)])
    plsc.subcore_barrier()

    gbase = wid * cpw

    def chunk(ci, carry):
        pltpu.sync_copy(dst_hbm.at[gbase + ci, 1], idx_v)
        pltpu.sync_copy(ones_v, deg_sh.at[idx_v], add=True)
        return carry

    lax.fori_loop(0, cpw, chunk, 0)
    plsc.subcore_barrier()
    pltpu.sync_copy(deg_sh.at[pl.ds(s * zs, zs)],
                    degp_hbm.at[pl.ds(c * n_pad + s * zs, zs)])


def _edge_kernel(n_pad, n0, n1, zs, d, ei_hbm, xs_hbm, outp_hbm,
                 ei_a, ei_b, rows_a, rows_b, z_v, out_sh, ga, gb, sa, sb):
    c = lax.axis_index("c")
    s = lax.axis_index("s")

    def zrow(i, carry):
        def zcol(j, cc):
            z_v[i, pl.ds(j * LANES, LANES)] = jnp.zeros((LANES,), jnp.float32)
            return cc

        lax.fori_loop(0, d // LANES, zcol, 0)
        return carry

    lax.fori_loop(0, 64, zrow, 0)

    def zcopy(t, carry):
        pltpu.sync_copy(z_v, out_sh.at[pl.ds(s * zs + t * 64, 64)])
        return carry

    lax.fori_loop(0, zs // 64, zcopy, 0)
    plsc.subcore_barrier()

    ncur = jnp.where(c == 0, n0, n1)
    gbase = c * NS * n0 + s * ncur

    def g_start(ei, rows, sem):
        pltpu.async_copy(xs_hbm.at[ei.at[0]], rows, sem)

    def g_wait(ei, rows, sem):
        pltpu.make_async_copy(xs_hbm.at[ei.at[0]], rows, sem).wait()

    def s_start(ei, rows, sem):
        pltpu.async_copy(rows, out_sh.at[ei.at[1]], sem, add=True)

    def s_wait(ei, rows, sem):
        pltpu.make_async_copy(rows, out_sh.at[ei.at[1]], sem).wait()

    pltpu.sync_copy(ei_hbm.at[gbase], ei_a)
    g_start(ei_a, rows_a, ga)
    pltpu.sync_copy(ei_hbm.at[gbase + 1], ei_b)
    g_start(ei_b, rows_b, gb)
    g_wait(ei_a, rows_a, ga)
    s_start(ei_a, rows_a, sa)

    def body(t, carry):
        g0 = gbase + 2 * t
        s_wait(ei_a, rows_a, sa)
        pltpu.sync_copy(ei_hbm.at[g0], ei_a)
        g_start(ei_a, rows_a, ga)
        g_wait(ei_b, rows_b, gb)
        s_start(ei_b, rows_b, sb)
        s_wait(ei_b, rows_b, sb)
        pltpu.sync_copy(ei_hbm.at[g0 + 1], ei_b)
        g_start(ei_b, rows_b, gb)
        g_wait(ei_a, rows_a, ga)
        s_start(ei_a, rows_a, sa)
        return carry

    lax.fori_loop(1, ncur // 2, body, 0)

    g_wait(ei_b, rows_b, gb)
    s_start(ei_b, rows_b, sb)
    s_wait(ei_a, rows_a, sa)
    s_wait(ei_b, rows_b, sb)

    plsc.subcore_barrier()
    pltpu.sync_copy(out_sh.at[pl.ds(s * zs, zs)],
                    outp_hbm.at[pl.ds(c * n_pad + s * zs, zs)])


def _dis_from_parts(dp_ref):
    deg = dp_ref[0, :] + dp_ref[1, :]
    return jnp.where(deg > 0, lax.rsqrt(jnp.maximum(deg, 1.0)), 0.0)


def _scale_kernel(dp_ref, x_ref, xs_ref):
    dis = _dis_from_parts(dp_ref)
    xs_ref[...] = x_ref[...] * dis[:, None]


def _combine_kernel(p_ref, dp_ref, o_ref):
    dis = _dis_from_parts(dp_ref)
    o_ref[...] = (p_ref[0] + p_ref[1]) * dis[:, None]


@jax.jit
def kernel(x, edge_index):
    n, d = x.shape
    e = edge_index.shape[1]

    n_pad = ((n + NS * LANES - 1) // (NS * LANES)) * (NS * LANES)
    zs = n_pad // NS
    cpw = -(-e // (NW * B))
    cpw += cpw % 2
    e_pad = cpw * B * NW
    sac = n_pad - 1

    total_per_pair = cpw * NC
    n0 = (total_per_pair * BIAS_NUM // BIAS_DEN) & ~1
    n1 = total_per_pair - n0

    ei = (
        jnp.full((2, e_pad), sac, jnp.int32)
        .at[:, :e].set(edge_index)
        .reshape(2, NW * cpw, B)
        .transpose(1, 0, 2)
    )
    x_pad = jnp.zeros((n_pad, d), x.dtype).at[:n].set(x)

    mesh = plsc.VectorSubcoreMesh(core_axis_name="c", subcore_axis_name="s",
                                  num_cores=NC, num_subcores=NS)

    deg_parts = pl.kernel(
        functools.partial(_deg_kernel, n_pad, cpw, zs),
        out_type=jax.ShapeDtypeStruct((NC * n_pad,), jnp.float32),
        mesh=mesh,
        scratch_types=[
            pltpu.VMEM((B,), jnp.int32),
            pltpu.VMEM((B,), jnp.float32),
            pltpu.VMEM((zs,), jnp.float32),
            pltpu.VMEM_SHARED((n_pad,), jnp.float32),
            pltpu.SemaphoreType.DMA,
        ],
    )(ei)
    deg_parts = deg_parts.reshape(NC, n_pad)

    rb = 1024
    grid = n_pad // rb
    xs = pl.pallas_call(
        _scale_kernel,
        grid=(grid,),
        in_specs=[
            pl.BlockSpec((NC, rb), lambda i: (0, i)),
            pl.BlockSpec((rb, d), lambda i: (i, 0)),
        ],
        out_specs=pl.BlockSpec((rb, d), lambda i: (i, 0)),
        out_shape=jax.ShapeDtypeStruct((n_pad, d), jnp.float32),
    )(deg_parts, x_pad)

    out_parts = pl.kernel(
        functools.partial(_edge_kernel, n_pad, n0, n1, zs, d),
        out_type=jax.ShapeDtypeStruct((NC * n_pad, d), jnp.float32),
        mesh=mesh,
        scratch_types=[
            pltpu.VMEM((2, B), jnp.int32),
            pltpu.VMEM((2, B), jnp.int32),
            pltpu.VMEM((B, d), jnp.float32),
            pltpu.VMEM((B, d), jnp.float32),
            pltpu.VMEM((64, d), jnp.float32),
            pltpu.VMEM_SHARED((n_pad, d), jnp.float32),
            pltpu.SemaphoreType.DMA,
            pltpu.SemaphoreType.DMA,
            pltpu.SemaphoreType.DMA,
            pltpu.SemaphoreType.DMA,
        ],
    )(ei, xs)
    out_parts = out_parts.reshape(NC, n_pad, d)

    out_pad = pl.pallas_call(
        _combine_kernel,
        grid=(grid,),
        in_specs=[
            pl.BlockSpec((NC, rb, d), lambda i: (0, i, 0)),
            pl.BlockSpec((NC, rb), lambda i: (0, i)),
        ],
        out_specs=pl.BlockSpec((rb, d), lambda i: (i, 0)),
        out_shape=jax.ShapeDtypeStruct((n_pad, d), jnp.float32),
    )(out_parts, deg_parts)

    return out_pad[:n]


# speedup vs baseline: 1.0774x; 1.0001x over previous
import functools

import jax
import jax.numpy as jnp
from jax import lax
from jax.experimental import pallas as pl
from jax.experimental.pallas import tpu as pltpu
from jax.experimental.pallas import tpu_sc as plsc

NC = 2
NS = 16
NW = NC * NS
LANES = 16
B = 128
BIAS_NUM = 63
BIAS_DEN = 80


def _fill_vec(ref, val, n):
    v = jnp.full((LANES,), val, dtype=ref.dtype)

    def body(i, c):
        ref[pl.ds(i * LANES, LANES)] = v
        return c

    lax.fori_loop(0, n // LANES, body, 0)


def _deg_kernel(n_pad, cpw, zs, dst_hbm, degp_hbm, idx_v, ones_v, z_v, deg_sh,
                sem):
    c = lax.axis_index("c")
    s = lax.axis_index("s")
    wid = s * NC + c

    _fill_vec(ones_v, 1.0, B)
    _fill_vec(z_v, 0.0, zs)
    pltpu.sync_copy(z_v, deg_sh.at[pl.ds(s * zs, zs)])
    plsc.subcore_barrier()

    gbase = wid * cpw

    def chunk(ci, carry):
        pltpu.sync_copy(dst_hbm.at[gbase + ci, 1], idx_v)
        pltpu.sync_copy(ones_v, deg_sh.at[idx_v], add=True)
        return carry

    lax.fori_loop(0, cpw, chunk, 0)
    plsc.subcore_barrier()
    pltpu.sync_copy(deg_sh.at[pl.ds(s * zs, zs)],
                    degp_hbm.at[pl.ds(c * n_pad + s * zs, zs)])


def _edge_kernel(n_pad, n0, n1, zs, d, ei_hbm, xs_hbm, outp_hbm,
                 ei_a, ei_b, rows_a, rows_b, z_v, out_sh, ga, gb, sa, sb):
    c = lax.axis_index("c")
    s = lax.axis_index("s")

    with jax.named_scope("acc_zero"):
        def zrow(i, carry):
            def zcol(j, cc):
                z_v[i, pl.ds(j * LANES, LANES)] = jnp.zeros((LANES,),
                                                            jnp.float32)
                return cc

            lax.fori_loop(0, d // LANES, zcol, 0)
            return carry

        lax.fori_loop(0, 64, zrow, 0)

        def zcopy(t, carry):
            pltpu.sync_copy(z_v, out_sh.at[pl.ds(s * zs + t * 64, 64)])
            return carry

        lax.fori_loop(0, zs // 64, zcopy, 0)
        plsc.subcore_barrier()

    ncur = jnp.where(c == 0, n0, n1)
    gbase = c * NS * n0 + s * ncur

    def g_start(ei, rows, sem):
        pltpu.async_copy(xs_hbm.at[ei.at[0]], rows, sem)

    def g_wait(ei, rows, sem):
        pltpu.make_async_copy(xs_hbm.at[ei.at[0]], rows, sem).wait()

    def s_start(ei, rows, sem):
        pltpu.async_copy(rows, out_sh.at[ei.at[1]], sem, add=True)

    def s_wait(ei, rows, sem):
        pltpu.make_async_copy(rows, out_sh.at[ei.at[1]], sem).wait()

    sc0 = jax.named_scope("chunk_loop")
    sc0.__enter__()
    pltpu.sync_copy(ei_hbm.at[gbase], ei_a)
    g_start(ei_a, rows_a, ga)
    pltpu.sync_copy(ei_hbm.at[gbase + 1], ei_b)
    g_start(ei_b, rows_b, gb)
    g_wait(ei_a, rows_a, ga)
    s_start(ei_a, rows_a, sa)

    def body(t, carry):
        g0 = gbase + 2 * t
        s_wait(ei_a, rows_a, sa)
        pltpu.sync_copy(ei_hbm.at[g0], ei_a)
        g_start(ei_a, rows_a, ga)
        g_wait(ei_b, rows_b, gb)
        s_start(ei_b, rows_b, sb)
        s_wait(ei_b, rows_b, sb)
        pltpu.sync_copy(ei_hbm.at[g0 + 1], ei_b)
        g_start(ei_b, rows_b, gb)
        g_wait(ei_a, rows_a, ga)
        s_start(ei_a, rows_a, sa)
        return carry

    lax.fori_loop(1, ncur // 2, body, 0)

    g_wait(ei_b, rows_b, gb)
    s_start(ei_b, rows_b, sb)
    s_wait(ei_a, rows_a, sa)
    s_wait(ei_b, rows_b, sb)
    sc0.__exit__(None, None, None)

    with jax.named_scope("copyout"):
        plsc.subcore_barrier()
        pltpu.sync_copy(out_sh.at[pl.ds(s * zs, zs)],
                        outp_hbm.at[pl.ds(c * n_pad + s * zs, zs)])


def _dis_from_parts(dp_ref):
    deg = dp_ref[0, :] + dp_ref[1, :]
    return jnp.where(deg > 0, lax.rsqrt(jnp.maximum(deg, 1.0)), 0.0)


def _scale_kernel(dp_ref, x_ref, xs_ref):
    dis = _dis_from_parts(dp_ref)
    xs_ref[...] = x_ref[...] * dis[:, None]


def _combine_kernel(p_ref, dp_ref, o_ref):
    dis = _dis_from_parts(dp_ref)
    o_ref[...] = (p_ref[0] + p_ref[1]) * dis[:, None]


@jax.jit
def kernel(x, edge_index):
    n, d = x.shape
    e = edge_index.shape[1]

    n_pad = ((n + NS * LANES - 1) // (NS * LANES)) * (NS * LANES)
    zs = n_pad // NS
    cpw = -(-e // (NW * B))
    cpw += cpw % 2
    e_pad = cpw * B * NW
    sac = n_pad - 1

    total_per_pair = cpw * NC
    n0 = (total_per_pair * BIAS_NUM // BIAS_DEN) & ~1
    n1 = total_per_pair - n0

    ei = (
        jnp.full((2, e_pad), sac, jnp.int32)
        .at[:, :e].set(edge_index)
        .reshape(2, NW * cpw, B)
        .transpose(1, 0, 2)
    )
    x_pad = jnp.zeros((n_pad, d), x.dtype).at[:n].set(x)

    mesh = plsc.VectorSubcoreMesh(core_axis_name="c", subcore_axis_name="s",
                                  num_cores=NC, num_subcores=NS)

    deg_parts = pl.kernel(
        functools.partial(_deg_kernel, n_pad, cpw, zs),
        out_type=jax.ShapeDtypeStruct((NC * n_pad,), jnp.float32),
        mesh=mesh,
        scratch_types=[
            pltpu.VMEM((B,), jnp.int32),
            pltpu.VMEM((B,), jnp.float32),
            pltpu.VMEM((zs,), jnp.float32),
            pltpu.VMEM_SHARED((n_pad,), jnp.float32),
            pltpu.SemaphoreType.DMA,
        ],
    )(ei)
    deg_parts = deg_parts.reshape(NC, n_pad)

    rb = 1024
    grid = n_pad // rb
    xs = pl.pallas_call(
        _scale_kernel,
        grid=(grid,),
        in_specs=[
            pl.BlockSpec((NC, rb), lambda i: (0, i)),
            pl.BlockSpec((rb, d), lambda i: (i, 0)),
        ],
        out_specs=pl.BlockSpec((rb, d), lambda i: (i, 0)),
        out_shape=jax.ShapeDtypeStruct((n_pad, d), jnp.float32),
    )(deg_parts, x_pad)

    out_parts = pl.kernel(
        functools.partial(_edge_kernel, n_pad, n0, n1, zs, d),
        out_type=jax.ShapeDtypeStruct((NC * n_pad, d), jnp.float32),
        mesh=mesh,
        scratch_types=[
            pltpu.VMEM((2, B), jnp.int32),
            pltpu.VMEM((2, B), jnp.int32),
            pltpu.VMEM((B, d), jnp.float32),
            pltpu.VMEM((B, d), jnp.float32),
            pltpu.VMEM((64, d), jnp.float32),
            pltpu.VMEM_SHARED((n_pad, d), jnp.float32),
            pltpu.SemaphoreType.DMA,
            pltpu.SemaphoreType.DMA,
            pltpu.SemaphoreType.DMA,
            pltpu.SemaphoreType.DMA,
        ],
    )(ei, xs)
    out_parts = out_parts.reshape(NC, n_pad, d)

    out_pad = pl.pallas_call(
        _combine_kernel,
        grid=(grid,),
        in_specs=[
            pl.BlockSpec((NC, rb, d), lambda i: (0, i, 0)),
            pl.BlockSpec((NC, rb), lambda i: (0, i)),
        ],
        out_specs=pl.BlockSpec((rb, d), lambda i: (i, 0)),
        out_shape=jax.ShapeDtypeStruct((n_pad, d), jnp.float32),
    )(out_parts, deg_parts)

    return out_pad[:n]
